# Initial kernel scaffold; baseline (speedup 1.0000x reference)
import jax
import jax.numpy as jnp
from jax.experimental import pallas as pl


def kernel(node_feat, edge, edge_feat, msg_W1, msg_b1, msg_W2, msg_b2, att_W1, att_b1, att_W2, att_b2, gru_Wih, gru_Whh, gru_bih, gru_bhh):
    raise NotImplementedError("write your pallas kernel here")


# speedup vs baseline: 2.7759x; 2.7759x over previous
import functools

import jax
import jax.numpy as jnp
from jax import lax
from jax.experimental import pallas as pl
from jax.experimental.pallas import tpu as pltpu
from jax.experimental.pallas import tpu_sc as plsc

_N = 10000
_E = 320000
_D = 128
_H = 128

_NC = 2
_NS = 16
_NW = _NC * _NS
_EPW = _E // _NW
_CH = 128
_NFULL = _EPW // _CH
_TAIL = _EPW - _NFULL * _CH
_RPS = 632
_NP = _RPS * _NS

_MESH = dict(core_axis_name="c", subcore_axis_name="s", num_cores=_NC,
             num_subcores=_NS)


def _gather_diff_body(state_h, nstate_h, src_h, dst_h, out_h,
                      sidx, didx, rows, sidx_t, didx_t, rows_t, sem):
  wid = lax.axis_index("s") * _NC + lax.axis_index("c")
  base0 = wid * _EPW

  def chunk(base, si, di, rw, k):
    pltpu.sync_copy(src_h.at[pl.ds(base, k)], si)
    pltpu.sync_copy(dst_h.at[pl.ds(base, k)], di)
    pltpu.async_copy(nstate_h.at[di], rw, sem).wait()
    pltpu.async_copy(state_h.at[si], rw, sem, add=True).wait()
    pltpu.sync_copy(rw, out_h.at[pl.ds(base, k)])

  def body(i, carry):
    chunk(base0 + i * _CH, sidx, didx, rows, _CH)
    return carry

  lax.fori_loop(0, _NFULL, body, 0)
  chunk(base0 + _NFULL * _CH, sidx_t, didx_t, rows_t, _TAIL)


def _make_gather_diff(interpret=False):
  return functools.partial(
      pl.kernel,
      out_type=jax.ShapeDtypeStruct((_E, _D), jnp.float32),
      mesh=plsc.VectorSubcoreMesh(**_MESH),
      scratch_types=[
          pltpu.VMEM((_CH,), jnp.int32),
          pltpu.VMEM((_CH,), jnp.int32),
          pltpu.VMEM((_CH, _D), jnp.float32),
          pltpu.VMEM((_TAIL,), jnp.int32),
          pltpu.VMEM((_TAIL,), jnp.int32),
          pltpu.VMEM((_TAIL, _D), jnp.float32),
          pltpu.SemaphoreType.DMA,
      ],
      interpret=interpret,
  )(_gather_diff_body)


def _scatter_body(msg_h, dst_h, zeros_h, out_h,
                  idx, rows, idx_t, rows_t, acc):
  c = lax.axis_index("c")
  s = lax.axis_index("s")
  wid = s * _NC + c
  r0 = s * _RPS
  pltpu.sync_copy(zeros_h.at[pl.ds(r0, _RPS)], acc.at[pl.ds(r0, _RPS)])
  plsc.subcore_barrier()

  base0 = wid * _EPW

  def body(i, carry):
    base = base0 + i * _CH
    pltpu.sync_copy(dst_h.at[pl.ds(base, _CH)], idx)
    pltpu.sync_copy(msg_h.at[pl.ds(base, _CH)], rows)
    pltpu.sync_copy(rows, acc.at[idx], add=True)
    return carry

  lax.fori_loop(0, _NFULL, body, 0)
  base = base0 + _NFULL * _CH
  pltpu.sync_copy(dst_h.at[pl.ds(base, _TAIL)], idx_t)
  pltpu.sync_copy(msg_h.at[pl.ds(base, _TAIL)], rows_t)
  pltpu.sync_copy(rows_t, acc.at[idx_t], add=True)

  plsc.subcore_barrier()
  pltpu.sync_copy(acc.at[pl.ds(r0, _RPS)], out_h.at[c, pl.ds(r0, _RPS)])


def _make_scatter(interpret=False):
  return functools.partial(
      pl.kernel,
      out_type=jax.ShapeDtypeStruct((_NC, _NP, _D), jnp.float32),
      mesh=plsc.VectorSubcoreMesh(**_MESH),
      scratch_types=[
          pltpu.VMEM((_CH,), jnp.int32),
          pltpu.VMEM((_CH, _D), jnp.float32),
          pltpu.VMEM((_TAIL,), jnp.int32),
          pltpu.VMEM((_TAIL, _D), jnp.float32),
          pltpu.VMEM_SHARED((_NP, _D), jnp.float32),
      ],
      interpret=interpret,
  )(_scatter_body)


_BM = 1000


def _mlp_body(diff_ref, ef_ref, mw1d, mw1e, mb1, mw2, mb2,
              aw1d, aw1e, ab1, aw2, ab2, out_ref):
  d = diff_ref[...]
  ef = ef_ref[...]
  h = jnp.maximum(jnp.dot(d, mw1d[...]) + jnp.dot(ef, mw1e[...]) + mb1[...],
                  0.0)
  m = jnp.dot(h, mw2[...]) + mb2[...]
  ha = jnp.maximum(jnp.dot(d, aw1d[...]) + jnp.dot(ef, aw1e[...]) + ab1[...],
                   0.0)
  a = jax.nn.sigmoid(jnp.dot(ha, aw2[...]) + ab2[...])
  out_ref[...] = m * a


def _mlp(diff, ef, mw1d, mw1e, mb1, mw2, mb2, aw1d, aw1e, ab1, aw2, ab2,
         interpret=False):
  full = pl.BlockSpec((_H, _H), lambda i: (0, 0))
  bias = pl.BlockSpec((1, _H), lambda i: (0, 0))
  blk = pl.BlockSpec((_BM, _H), lambda i: (i, 0))
  return pl.pallas_call(
      _mlp_body,
      grid=(_E // _BM,),
      in_specs=[blk, blk, full, full, bias, full, bias,
                full, full, bias, full, bias],
      out_specs=blk,
      out_shape=jax.ShapeDtypeStruct((_E, _H), jnp.float32),
      interpret=interpret,
  )(diff, ef, mw1d, mw1e, mb1, mw2, mb2, aw1d, aw1e, ab1, aw2, ab2)


_BN = 1000


def _gru_body(p_ref, st_ref, wih, whh, bih, bhh, out_ref):
  sm = p_ref[0] + p_ref[1]
  st = st_ref[...]
  gi = jnp.dot(sm, wih[...]) + bih[...]
  gh = jnp.dot(st, whh[...]) + bhh[...]
  r = jax.nn.sigmoid(gi[:, :_H] + gh[:, :_H])
  z = jax.nn.sigmoid(gi[:, _H:2 * _H] + gh[:, _H:2 * _H])
  n = jnp.tanh(gi[:, 2 * _H:] + r * gh[:, 2 * _H:])
  out_ref[...] = (1.0 - z) * n + z * st


def _gru(partials, state, wihT, whhT, bih, bhh, interpret=False):
  return pl.pallas_call(
      _gru_body,
      grid=(_N // _BN,),
      in_specs=[
          pl.BlockSpec((_NC, _BN, _H), lambda i: (0, i, 0)),
          pl.BlockSpec((_BN, _H), lambda i: (i, 0)),
          pl.BlockSpec((_H, 3 * _H), lambda i: (0, 0)),
          pl.BlockSpec((_H, 3 * _H), lambda i: (0, 0)),
          pl.BlockSpec((1, 3 * _H), lambda i: (0, 0)),
          pl.BlockSpec((1, 3 * _H), lambda i: (0, 0)),
      ],
      out_specs=pl.BlockSpec((_BN, _H), lambda i: (i, 0)),
      out_shape=jax.ShapeDtypeStruct((_N, _H), jnp.float32),
      interpret=interpret,
  )(partials, state, wihT, whhT, bih, bhh)


def kernel(node_feat, edge, edge_feat, msg_W1, msg_b1, msg_W2, msg_b2,
           att_W1, att_b1, att_W2, att_b2, gru_Wih, gru_Whh, gru_bih,
           gru_bhh):
  src = edge[:, 0]
  dst = edge[:, 1]
  nstate = -node_feat

  diff = _make_gather_diff()(node_feat, nstate, src, dst)

  msg = _mlp(
      diff, edge_feat,
      msg_W1[:, :_D].T, msg_W1[:, _D:].T, msg_b1.reshape(1, _H),
      msg_W2.T, msg_b2.reshape(1, _H),
      att_W1[:, :_D].T, att_W1[:, _D:].T, att_b1.reshape(1, _H),
      att_W2.T, att_b2.reshape(1, _H))

  partials = _make_scatter()(msg, dst, jnp.zeros((_NP, _D), jnp.float32))

  return _gru(partials, node_feat, gru_Wih.T, gru_Whh.T,
              gru_bih.reshape(1, 3 * _H), gru_bhh.reshape(1, 3 * _H))
